# baseline (speedup 1.0000x reference)
import functools

import jax
import jax.numpy as jnp
import numpy as np
from jax import lax
from jax.experimental import pallas as pl
from jax.experimental.pallas import tpu as pltpu
from jax.experimental.pallas import tpu_sc as plsc

H = 768
NLANE = 16
NCOL = H // NLANE
N_PAGES = 16
SEQ = 512
BODY = 502
NPT = 10
GRID = 14
IMG_SIDE = 224
NVIS = GRID * GRID
ROW_OUT = BODY + NVIS + NPT
CH = 8
NCH = 256 // CH
VCH = 98 // CH


def _pe_table():
    n, d = 4, H
    pos = np.arange(n)[:, None].astype(np.float32)
    i = np.arange(d)[None, :].astype(np.float32)
    angle = pos / np.power(10000.0, (2.0 * np.floor(i / 2.0)) / d)
    pe = np.zeros((n, d), dtype=np.float32)
    pe[:, 0::2] = np.sin(angle[:, 0::2])
    pe[:, 1::2] = np.cos(angle[:, 1::2])
    return pe


def _grid_idx():
    xs = (np.arange(GRID) * 1000) // GRID
    xe = ((np.arange(GRID) + 1) * 1000) // GRID
    return np.pad(np.concatenate([xs, xe]), (0, 4)).astype(np.int32)


def _visual_matmul(patches, W, b):
    def body(a_ref, w_ref, b_ref, o_ref):
        o_ref[...] = jnp.dot(a_ref[...], w_ref[...],
                             preferred_element_type=jnp.float32) + b_ref[...]

    return pl.pallas_call(
        body,
        grid=(8,),
        in_specs=[pl.BlockSpec((392, 256), lambda i: (i, 0)),
                  pl.BlockSpec((256, H), lambda i: (0, 0)),
                  pl.BlockSpec((1, H), lambda i: (0, 0))],
        out_specs=pl.BlockSpec((392, H), lambda i: (i, 0)),
        out_shape=jax.ShapeDtypeStruct((N_PAGES * NVIS, H), jnp.float32),
    )(patches, W, b.reshape(1, H))


def _sc_embed(shared, x_t, y_t, vd, pe, ids, xi, yi, g_i):
    mesh = plsc.VectorSubcoreMesh(core_axis_name="c", subcore_axis_name="s")

    @functools.partial(
        pl.kernel, mesh=mesh,
        out_type=jax.ShapeDtypeStruct((N_PAGES * ROW_OUT, H), jnp.float32),
        compiler_params=pltpu.CompilerParams(use_tc_tiling_on_sc=False),
        scratch_types=[
            pltpu.VMEM((256,), jnp.int32),
            pltpu.VMEM((512,), jnp.int32),
            pltpu.VMEM((512,), jnp.int32),
            pltpu.VMEM((32,), jnp.int32),
            pltpu.VMEM((CH, H), jnp.float32),
            pltpu.VMEM((CH, H), jnp.float32),
            pltpu.VMEM((2 * CH, H), jnp.float32),
            pltpu.VMEM((2 * CH, H), jnp.float32),
            pltpu.VMEM((2 * CH, H), jnp.float32),
            pltpu.VMEM((2 * CH, H), jnp.float32),
            pltpu.VMEM((28, H), jnp.float32),
            pltpu.VMEM((28, H), jnp.float32),
            pltpu.VMEM((1, H), jnp.float32),
            pltpu.SemaphoreType.DMA,
            pltpu.SemaphoreType.DMA,
            pltpu.SemaphoreType.DMA,
            pltpu.SemaphoreType.DMA,
            pltpu.SemaphoreType.DMA,
        ],
    )
    def k(shared_h, xt_h, yt_h, vd_h, pe_h, ids_h, xi_h, yi_h, g_h,
          out_h,
          ids_v, xi_v, yi_v, g_v,
          s0, s1, x0, x1, y0, y1, bgx, bgy, pe_v,
          sem_g0, sem_g1, sem_s0, sem_s1, sem_c):
        n = lax.axis_index("s")
        h = lax.axis_index("c")
        tok0 = n * SEQ + h * 256
        out0 = n * ROW_OUT + h * 256

        S = (s0, s1)
        X = (x0, x1)
        Y = (y0, y1)
        SEM_G = (sem_g0, sem_g1)
        SEM_S = (sem_s0, sem_s1)

        pltpu.sync_copy(ids_h.at[pl.ds(tok0, 256)], ids_v)
        pltpu.sync_copy(xi_h.at[pl.ds(2 * tok0, 512)], xi_v)
        pltpu.sync_copy(yi_h.at[pl.ds(2 * tok0, 512)], yi_v)
        pltpu.sync_copy(g_h, g_v)
        pltpu.sync_copy(pe_h.at[pl.ds(n % 4, 1)], pe_v)

        pltpu.async_copy(xt_h.at[g_v.at[pl.ds(0, 28)]], bgx, sem_c)
        pltpu.async_copy(yt_h.at[g_v.at[pl.ds(0, 28)]], bgy, sem_c)

        def fire(ci, p):
            off = ci * CH
            pltpu.async_copy(shared_h.at[ids_v.at[pl.ds(off, CH)]],
                             S[p], SEM_G[p])
            pltpu.async_copy(xt_h.at[xi_v.at[pl.ds(2 * off, 2 * CH)]],
                             X[p], SEM_G[p])
            pltpu.async_copy(yt_h.at[yi_v.at[pl.ds(2 * off, 2 * CH)]],
                             Y[p], SEM_G[p])

        def wait_g(p):
            pltpu.make_async_copy(shared_h.at[pl.ds(0, CH)], S[p], SEM_G[p]).wait()
            pltpu.make_async_copy(xt_h.at[pl.ds(0, 2 * CH)], X[p], SEM_G[p]).wait()
            pltpu.make_async_copy(yt_h.at[pl.ds(0, 2 * CH)], Y[p], SEM_G[p]).wait()

        def wait_st(p):
            pltpu.make_async_copy(S[p], out_h.at[pl.ds(0, CH)], SEM_S[p]).wait()

        def tok_sum(p):
            sp, xp, yp = S[p], X[p], Y[p]

            def rbody(r, carry):
                for g in range(NCOL):
                    sl = pl.ds(g * NLANE, NLANE)
                    acc = sp[r, sl] + xp[2 * r, sl]
                    acc = acc + xp[2 * r + 1, sl]
                    acc = acc + yp[2 * r, sl]
                    sp[r, sl] = acc + yp[2 * r + 1, sl]
                return carry
            lax.fori_loop(0, CH, rbody, 0)

        def pe_add(p, lo, hi):
            sp = S[p]

            def rbody(r, carry):
                for g in range(NCOL):
                    sl = pl.ds(g * NLANE, NLANE)
                    sp[r, sl] = sp[r, sl] + pe_v[0, sl]
                return carry
            lax.fori_loop(lo, hi, rbody, 0)

        fire(0, 0)
        fire(1, 1)

        def pair(i2, carry):
            ci = 2 * i2

            def stage(p, cc):
                wait_g(p)
                tok_sum(p)
                pltpu.async_copy(S[p], out_h.at[pl.ds(out0 + cc * CH, CH)],
                                 SEM_S[p])
            stage(0, ci)
            stage(1, ci + 1)
            wait_st(0)
            fire(ci + 2, 0)
            wait_st(1)
            fire(ci + 3, 1)
            return carry
        lax.fori_loop(0, (NCH - 2) // 2, pair, 0)

        wait_g(0)
        tok_sum(0)

        @pl.when(h == 0)
        def _():
            pltpu.async_copy(S[0], out_h.at[pl.ds(out0 + 240, CH)], SEM_S[0])

        @pl.when(h == 1)
        def _():
            pe_add(0, 6, 8)
            pltpu.async_copy(S[0].at[pl.ds(0, 6)],
                             out_h.at[pl.ds(n * ROW_OUT + 496, 6)], SEM_S[0])
            pltpu.async_copy(S[0].at[pl.ds(6, 2)],
                             out_h.at[pl.ds(n * ROW_OUT + BODY + NVIS, 2)],
                             SEM_S[0])

        wait_g(1)
        tok_sum(1)

        @pl.when(h == 0)
        def _():
            pltpu.async_copy(S[1], out_h.at[pl.ds(out0 + 248, CH)], SEM_S[1])

        @pl.when(h == 1)
        def _():
            pe_add(1, 0, 8)
            pltpu.async_copy(S[1],
                             out_h.at[pl.ds(n * ROW_OUT + BODY + NVIS + 2, CH)],
                             SEM_S[1])

        pltpu.make_async_copy(xt_h.at[pl.ds(0, 28)], bgx, sem_c).wait()
        pltpu.make_async_copy(yt_h.at[pl.ds(0, 28)], bgy, sem_c).wait()
        vbase = n * NVIS + h * 98
        obase = n * ROW_OUT + BODY + h * 98

        def vis_fire(ci, p):
            pltpu.async_copy(vd_h.at[pl.ds(vbase + ci * CH, CH)], S[p],
                             SEM_G[p])

        def vis_wait_g(p):
            pltpu.make_async_copy(vd_h.at[pl.ds(0, CH)], S[p], SEM_G[p]).wait()

        def vis_sum(p, ci):
            sp = S[p]
            rg0 = h * 98 + ci * CH

            def rbody(r, carry):
                rg = rg0 + r
                ii = rg // GRID
                jj = rg - ii * GRID
                for g in range(NCOL):
                    sl = pl.ds(g * NLANE, NLANE)
                    acc = sp[r, sl] + bgx[jj, sl]
                    acc = acc + bgx[14 + jj, sl]
                    acc = acc + bgy[ii, sl]
                    sp[r, sl] = acc + bgy[14 + ii, sl]
                return carry
            lax.fori_loop(0, CH, rbody, 0)

        wait_st(0)
        vis_fire(0, 0)
        wait_st(1)
        vis_fire(1, 1)

        def vpair(i2, carry):
            ci = 2 * i2

            def stage(p, cc):
                vis_wait_g(p)
                vis_sum(p, cc)
                pltpu.async_copy(S[p], out_h.at[pl.ds(obase + cc * CH, CH)],
                                 SEM_S[p])
            stage(0, ci)
            stage(1, ci + 1)
            wait_st(0)
            pass_p0 = ci + 2
            pass_p1 = ci + 3

            @pl.when(pass_p0 < VCH)
            def _():
                vis_fire(pass_p0, 0)
            wait_st(1)

            @pl.when(pass_p1 < VCH)
            def _():
                vis_fire(pass_p1, 1)
            return carry
        lax.fori_loop(0, VCH // 2, vpair, 0)

        pltpu.async_copy(vd_h.at[pl.ds(vbase + 96, 2)], S[0].at[pl.ds(0, 2)],
                         SEM_G[0])
        pltpu.make_async_copy(vd_h.at[pl.ds(0, 2)], S[0].at[pl.ds(0, 2)],
                              SEM_G[0]).wait()

        def tbody(r, carry):
            rg = h * 98 + 96 + r
            ii = rg // GRID
            jj = rg - ii * GRID
            for g in range(NCOL):
                sl = pl.ds(g * NLANE, NLANE)
                acc = S[0][r, sl] + bgx[jj, sl]
                acc = acc + bgx[14 + jj, sl]
                acc = acc + bgy[ii, sl]
                S[0][r, sl] = acc + bgy[14 + ii, sl]
            return carry
        lax.fori_loop(0, 2, tbody, 0)
        pltpu.sync_copy(S[0].at[pl.ds(0, 2)], out_h.at[pl.ds(obase + 96, 2)])

    return k(shared, x_t, y_t, vd, pe, ids, xi, yi, g_i)


def kernel(input_ids, boxes, images, shared, x_table, y_table, W_vis, b_vis):
    ids = input_ids.reshape(-1).astype(jnp.int32)
    bf = boxes.reshape(-1, 4).astype(jnp.int32)
    xi = bf[:, (0, 2)].reshape(-1)
    yi = bf[:, (1, 3)].reshape(-1)
    imgs = images.reshape(-1, IMG_SIDE, IMG_SIDE)
    patches = imgs.reshape(-1, GRID, 16, GRID, 16).transpose(0, 1, 3, 2, 4)
    patches = patches.reshape(N_PAGES * NVIS, 256)
    vd = _visual_matmul(patches, W_vis, b_vis)
    pe = jnp.asarray(_pe_table())
    g_i = jnp.asarray(_grid_idx())
    out = _sc_embed(shared, x_table, y_table, vd, pe, ids, xi, yi, g_i)
    return out.reshape(N_PAGES, ROW_OUT, H)


# speedup vs baseline: 1.2597x; 1.2597x over previous
import functools

import jax
import jax.numpy as jnp
import numpy as np
from jax import lax
from jax.experimental import pallas as pl
from jax.experimental.pallas import tpu as pltpu
from jax.experimental.pallas import tpu_sc as plsc

H = 768
NLANE = 16
NCOL = H // NLANE
N_PAGES = 16
SEQ = 512
BODY = 502
NPT = 10
GRID = 14
IMG_SIDE = 224
NVIS = GRID * GRID
ROW_OUT = BODY + NVIS + NPT
CH = 8
NCH = 256 // CH


def _pe_table():
    n, d = 4, H
    pos = np.arange(n)[:, None].astype(np.float32)
    i = np.arange(d)[None, :].astype(np.float32)
    angle = pos / np.power(10000.0, (2.0 * np.floor(i / 2.0)) / d)
    pe = np.zeros((n, d), dtype=np.float32)
    pe[:, 0::2] = np.sin(angle[:, 0::2])
    pe[:, 1::2] = np.cos(angle[:, 1::2])
    return pe


def _tail_rg(r):
    return 90 + ((r + 6) % 8)


def _grid_vis_idx():
    xs = (np.arange(GRID) * 1000) // GRID
    xe = ((np.arange(GRID) + 1) * 1000) // GRID
    vxi = np.zeros((32, 256), dtype=np.int32)
    vyi = np.zeros((32, 256), dtype=np.int32)
    for w in range(32):
        h = w % 2
        for wi in range(12):
            for r in range(CH):
                rg = h * 98 + wi * CH + r
                ii, jj = rg // GRID, rg % GRID
                vxi[w, wi * 16 + 2 * r] = xs[jj]
                vxi[w, wi * 16 + 2 * r + 1] = xe[jj]
                vyi[w, wi * 16 + 2 * r] = xs[ii]
                vyi[w, wi * 16 + 2 * r + 1] = xe[ii]
        for r in range(CH):
            rg = h * 98 + _tail_rg(r)
            ii, jj = rg // GRID, rg % GRID
            vxi[w, 192 + 2 * r] = xs[jj]
            vxi[w, 192 + 2 * r + 1] = xe[jj]
            vyi[w, 192 + 2 * r] = xs[ii]
            vyi[w, 192 + 2 * r + 1] = xe[ii]
    return vxi, vyi


def _out_row_idx():
    oidx = np.zeros((32, 512), dtype=np.int32)
    tidx = np.zeros((32, 8, 8), dtype=np.int32)
    for w in range(32):
        n, h = w // 2, w % 2
        obase = n * ROW_OUT + BODY + h * 98
        for j in range(16):
            for k in range(16):
                s = h * 256 + j * 16 + k
                oidx[w, j * 16 + k] = (n * ROW_OUT + s if s < BODY
                                       else n * ROW_OUT + NVIS + s)
        for j2 in range(6):
            for k in range(16):
                oidx[w, 256 + j2 * 16 + k] = obase + j2 * 16 + k
        for r in range(CH):
            tidx[w, 0, r] = obase + _tail_rg(r)
    return oidx, tidx


def _visual_matmul(patches_pad, W, b):
    def body(a_ref, w_ref, b_ref, o_ref):
        o_ref[...] = jnp.dot(a_ref[0], w_ref[...],
                             preferred_element_type=jnp.float32)[None] + b_ref[...]

    return pl.pallas_call(
        body,
        grid=(32,),
        in_specs=[pl.BlockSpec((1, 104, 256), lambda i: (i, 0, 0)),
                  pl.BlockSpec((256, H), lambda i: (0, 0)),
                  pl.BlockSpec((1, 1, H), lambda i: (0, 0, 0))],
        out_specs=pl.BlockSpec((1, 104, H), lambda i: (i, 0, 0)),
        out_shape=jax.ShapeDtypeStruct((32, 104, H), jnp.float32),
    )(patches_pad, W, b.reshape(1, 1, H))


def _sc_embed(shared, x_t, y_t, vd, pe, ids, xi, yi, vxi, vyi, oidx, tidx,
              pe_i):
    mesh = plsc.VectorSubcoreMesh(core_axis_name="c", subcore_axis_name="s")

    @functools.partial(
        pl.kernel, mesh=mesh,
        out_type=jax.ShapeDtypeStruct((N_PAGES * ROW_OUT, H), jnp.float32),
        compiler_params=pltpu.CompilerParams(use_tc_tiling_on_sc=True),
        scratch_types=[
            pltpu.VMEM((256,), jnp.int32),
            pltpu.VMEM((768,), jnp.int32),
            pltpu.VMEM((768,), jnp.int32),
            pltpu.VMEM((512,), jnp.int32),
            pltpu.VMEM((8, 8), jnp.int32),
            pltpu.VMEM((8,), jnp.int32),
            pltpu.VMEM((CH, H), jnp.float32),
            pltpu.VMEM((CH, H), jnp.float32),
            pltpu.VMEM((2 * CH, H), jnp.float32),
            pltpu.VMEM((2 * CH, H), jnp.float32),
            pltpu.VMEM((2 * CH, H), jnp.float32),
            pltpu.VMEM((2 * CH, H), jnp.float32),
            pltpu.VMEM((2 * CH, H), jnp.float32),
            pltpu.VMEM((4, H), jnp.float32),
            pltpu.SemaphoreType.DMA,
            pltpu.SemaphoreType.DMA,
            pltpu.SemaphoreType.DMA,
            pltpu.SemaphoreType.DMA,
        ],
    )
    def k(shared_h, xt_h, yt_h, vd_h, pe_h, ids_h, xi_h, yi_h, vxi_h, vyi_h,
          oidx_h, tidx_h, pei_h,
          out_h,
          ids_v, xi_v, yi_v, oidx_v, tidx_v, pe_idx_v,
          s0, s1, x0, x1, y0, y1, ob, pe_v,
          sem_g0, sem_g1, sem_s, sem_c):
        n = lax.axis_index("s")
        h = lax.axis_index("c")
        w = n * 2 + h
        tok0 = n * SEQ + h * 256
        p4 = n % 4

        S = (s0, s1)
        X = (x0, x1)
        Y = (y0, y1)
        SEM_G = (sem_g0, sem_g1)

        pltpu.sync_copy(ids_h.at[pl.ds(tok0, 256)], ids_v)
        pltpu.sync_copy(xi_h.at[pl.ds(2 * tok0, 512)], xi_v.at[pl.ds(0, 512)])
        pltpu.sync_copy(yi_h.at[pl.ds(2 * tok0, 512)], yi_v.at[pl.ds(0, 512)])
        pltpu.sync_copy(vxi_h.at[w], xi_v.at[pl.ds(512, 256)])
        pltpu.sync_copy(vyi_h.at[w], yi_v.at[pl.ds(512, 256)])
        pltpu.sync_copy(oidx_h.at[w], oidx_v)
        pltpu.sync_copy(tidx_h.at[w], tidx_v)
        pltpu.sync_copy(pei_h, pe_idx_v)

        vg = pltpu.async_copy(pe_h.at[pe_idx_v.at[pl.ds(0, 4)]], pe_v, sem_c)

        def fire(ci, p):
            off = ci * CH
            pltpu.async_copy(shared_h.at[ids_v.at[pl.ds(off, CH)]],
                             S[p], SEM_G[p])
            pltpu.async_copy(xt_h.at[xi_v.at[pl.ds(2 * off, 2 * CH)]],
                             X[p], SEM_G[p])
            pltpu.async_copy(yt_h.at[yi_v.at[pl.ds(2 * off, 2 * CH)]],
                             Y[p], SEM_G[p])

        def vis_fire(wi, p):
            pltpu.async_copy(vd_h.at[w, pl.ds(wi * CH, CH)], S[p], SEM_G[p])
            pltpu.async_copy(xt_h.at[xi_v.at[pl.ds(512 + wi * 16, 16)]],
                             X[p], SEM_G[p])
            pltpu.async_copy(yt_h.at[yi_v.at[pl.ds(512 + wi * 16, 16)]],
                             Y[p], SEM_G[p])

        def wait_g(p):
            pltpu.make_async_copy(shared_h.at[pl.ds(0, CH)], S[p],
                                  SEM_G[p]).wait()
            pltpu.make_async_copy(xt_h.at[pl.ds(0, 2 * CH)], X[p],
                                  SEM_G[p]).wait()
            pltpu.make_async_copy(yt_h.at[pl.ds(0, 2 * CH)], Y[p],
                                  SEM_G[p]).wait()

        def wait_st():
            pltpu.make_async_copy(ob, out_h.at[pl.ds(0, 2 * CH)],
                                  sem_s).wait()

        def sum_rows(p):
            sp, xp, yp = S[p], X[p], Y[p]

            def rbody(r, carry):
                def load_pair(g):
                    sl0 = pl.ds(g * NLANE, NLANE)
                    sl1 = pl.ds((g + 1) * NLANE, NLANE)
                    return (sp[r, sl0], sp[r, sl1],
                            xp[2 * r, sl0], xp[2 * r, sl1],
                            xp[2 * r + 1, sl0], xp[2 * r + 1, sl1],
                            yp[2 * r, sl0], yp[2 * r, sl1],
                            yp[2 * r + 1, sl0], yp[2 * r + 1, sl1])

                def do_adds(g, t):
                    s_0, s_1, xa0, xa1, xb0, xb1, ya0, ya1, yb0, yb1 = t
                    sl0 = pl.ds(g * NLANE, NLANE)
                    sl1 = pl.ds((g + 1) * NLANE, NLANE)
                    a0 = s_0 + xa0
                    a1 = s_1 + xa1
                    b0_ = xb0 + ya0
                    b1_ = xb1 + ya1
                    ob[p * CH + r, sl0] = (a0 + yb0) + b0_
                    ob[p * CH + r, sl1] = (a1 + yb1) + b1_

                t = load_pair(0)
                for g in range(0, NCOL, 2):
                    nt = load_pair(g + 2) if g + 2 < NCOL else None
                    do_adds(g, t)
                    t = nt
                return carry
            lax.fori_loop(0, CH, rbody, 0)

        def pe_add(lo, hi):
            def rbody(r, carry):
                for g in range(NCOL):
                    sl = pl.ds(g * NLANE, NLANE)
                    ob[r, sl] = ob[r, sl] + pe_v[p4, sl]
                return carry
            lax.fori_loop(lo, hi, rbody, 0)

        def store_pair(idx_off):
            reg = oidx_v[pl.ds(idx_off, 16)]
            pltpu.async_copy(ob, out_h.at[reg], sem_s)

        fire(0, 0)
        fire(1, 1)

        def pair(j, carry):
            wait_g(0)
            pl.when(j >= 1)(lambda: wait_st())
            sum_rows(0)
            pl.when(j < 15)(lambda: fire(2 * j + 2, 0))
            wait_g(1)
            sum_rows(1)
            pl.when(j < 15)(lambda: fire(2 * j + 3, 1))
            pl.when(jnp.logical_and(j == 15, h == 1))(lambda: pe_add(6, 16))
            store_pair(j * 16)
            return carry
        lax.fori_loop(0, 16, pair, 0)

        vg.wait()
        wait_st()
        vis_fire(0, 0)
        vis_fire(1, 1)

        def vpair(j, carry):
            wait_g(0)
            pl.when(j >= 1)(lambda: wait_st())
            sum_rows(0)
            pl.when(j < 5)(lambda: vis_fire(2 * j + 2, 0))
            wait_g(1)
            sum_rows(1)
            pl.when(j < 5)(lambda: vis_fire(2 * j + 3, 1))
            store_pair(256 + j * 16)
            return carry
        lax.fori_loop(0, 6, vpair, 0)

        pltpu.async_copy(vd_h.at[w, pl.ds(88, CH)], S[0], SEM_G[0])
        pltpu.async_copy(vd_h.at[w, pl.ds(96, CH)], S[1], SEM_G[1])
        pltpu.async_copy(xt_h.at[xi_v.at[pl.ds(704, 16)]], X[0], SEM_G[0])
        pltpu.async_copy(yt_h.at[yi_v.at[pl.ds(704, 16)]], Y[0], SEM_G[0])
        pltpu.make_async_copy(vd_h.at[0, pl.ds(0, CH)], S[0], SEM_G[0]).wait()
        pltpu.make_async_copy(vd_h.at[0, pl.ds(0, CH)], S[1], SEM_G[1]).wait()
        pltpu.make_async_copy(xt_h.at[pl.ds(0, 2 * CH)], X[0], SEM_G[0]).wait()
        pltpu.make_async_copy(yt_h.at[pl.ds(0, 2 * CH)], Y[0], SEM_G[0]).wait()
        wait_st()

        for r in range(CH):
            rg = _tail_rg(r)
            dsrc = s1 if rg >= 96 else s0
            drow = rg - 96 if rg >= 96 else rg - 88
            for g in range(0, NCOL, 2):
                sl0 = pl.ds(g * NLANE, NLANE)
                sl1 = pl.ds((g + 1) * NLANE, NLANE)
                a0 = dsrc[drow, sl0] + x0[2 * r, sl0]
                a1 = dsrc[drow, sl1] + x0[2 * r, sl1]
                b0_ = x0[2 * r + 1, sl0] + y0[2 * r, sl0]
                b1_ = x0[2 * r + 1, sl1] + y0[2 * r, sl1]
                ob[r, sl0] = (a0 + y0[2 * r + 1, sl0]) + b0_
                ob[r, sl1] = (a1 + y0[2 * r + 1, sl1]) + b1_

        pltpu.async_copy(ob.at[pl.ds(0, CH)], out_h.at[tidx_v.at[0]], sem_s)
        pltpu.make_async_copy(ob.at[pl.ds(0, CH)], out_h.at[pl.ds(0, CH)],
                              sem_s).wait()

    return k(shared, x_t, y_t, vd, pe, ids, xi, yi, vxi, vyi, oidx, tidx,
             pe_i)


def kernel(input_ids, boxes, images, shared, x_table, y_table, W_vis, b_vis):
    ids = input_ids.reshape(-1).astype(jnp.int32)
    bf = boxes.reshape(-1, 4).astype(jnp.int32)
    xi = bf[:, (0, 2)].reshape(-1)
    yi = bf[:, (1, 3)].reshape(-1)
    imgs = images.reshape(-1, IMG_SIDE, IMG_SIDE)
    patches = imgs.reshape(-1, GRID, 16, GRID, 16).transpose(0, 1, 3, 2, 4)
    patches = patches.reshape(32, 98, 256)
    patches_pad = jnp.pad(patches, ((0, 0), (0, 6), (0, 0)))
    vd = _visual_matmul(patches_pad, W_vis, b_vis)
    pe = jnp.asarray(_pe_table())
    vxi, vyi = _grid_vis_idx()
    oidx, tidx = _out_row_idx()
    pe_i = jnp.asarray(np.arange(8, dtype=np.int32))
    out = _sc_embed(shared, x_table, y_table, vd, pe, ids, xi, yi,
                    jnp.asarray(vxi), jnp.asarray(vyi), jnp.asarray(oidx),
                    jnp.asarray(tidx), pe_i)
    return out.reshape(N_PAGES, ROW_OUT, H)


# speedup vs baseline: 1.3775x; 1.0935x over previous
import functools

import jax
import jax.numpy as jnp
import numpy as np
from jax import lax
from jax.experimental import pallas as pl
from jax.experimental.pallas import tpu as pltpu
from jax.experimental.pallas import tpu_sc as plsc

H = 768
NLANE = 16
NCOL = H // NLANE
N_PAGES = 16
SEQ = 512
BODY = 502
NPT = 10
GRID = 14
IMG_SIDE = 224
NVIS = GRID * GRID
ROW_OUT = BODY + NVIS + NPT
CH = 8
NCH = 256 // CH


def _pe_table():
    n, d = 4, H
    pos = np.arange(n)[:, None].astype(np.float32)
    i = np.arange(d)[None, :].astype(np.float32)
    angle = pos / np.power(10000.0, (2.0 * np.floor(i / 2.0)) / d)
    pe = np.zeros((n, d), dtype=np.float32)
    pe[:, 0::2] = np.sin(angle[:, 0::2])
    pe[:, 1::2] = np.cos(angle[:, 1::2])
    return pe


def _tail_rg(r):
    return 90 + ((r + 6) % 8)


def _grid_vis_idx():
    xs = (np.arange(GRID) * 1000) // GRID
    xe = ((np.arange(GRID) + 1) * 1000) // GRID
    vxi = np.zeros((32, 256), dtype=np.int32)
    vyi = np.zeros((32, 256), dtype=np.int32)
    for w in range(32):
        h = w % 2
        for wi in range(12):
            for r in range(CH):
                rg = h * 98 + wi * CH + r
                ii, jj = rg // GRID, rg % GRID
                vxi[w, wi * 16 + 2 * r] = xs[jj]
                vxi[w, wi * 16 + 2 * r + 1] = xe[jj]
                vyi[w, wi * 16 + 2 * r] = xs[ii]
                vyi[w, wi * 16 + 2 * r + 1] = xe[ii]
        for r in range(CH):
            rg = h * 98 + _tail_rg(r)
            ii, jj = rg // GRID, rg % GRID
            vxi[w, 192 + 2 * r] = xs[jj]
            vxi[w, 192 + 2 * r + 1] = xe[jj]
            vyi[w, 192 + 2 * r] = xs[ii]
            vyi[w, 192 + 2 * r + 1] = xe[ii]
    return vxi, vyi


def _out_row_idx():
    oidx = np.zeros((32, 512), dtype=np.int32)
    tidx = np.zeros((32, 8, 8), dtype=np.int32)
    for w in range(32):
        n, h = w // 2, w % 2
        obase = n * ROW_OUT + BODY + h * 98
        for j in range(16):
            for k in range(16):
                s = h * 256 + j * 16 + k
                oidx[w, j * 16 + k] = (n * ROW_OUT + s if s < BODY
                                       else n * ROW_OUT + NVIS + s)
        for j2 in range(6):
            for k in range(16):
                oidx[w, 256 + j2 * 16 + k] = obase + j2 * 16 + k
        for r in range(CH):
            tidx[w, 0, r] = obase + _tail_rg(r)
    return oidx, tidx


def _visual_add(out_sc, patches, W, b):
    def body(o_in, a_ref, w_ref, b_ref, o_ref):
        o_ref[...] = o_in[...]
        m = jnp.dot(a_ref[0], w_ref[...],
                    preferred_element_type=jnp.float32) + b_ref[0]
        o_ref[0, pl.ds(BODY, NVIS), :] = o_in[0, pl.ds(BODY, NVIS), :] + m

    return pl.pallas_call(
        body,
        grid=(N_PAGES,),
        in_specs=[pl.BlockSpec((1, ROW_OUT, H), lambda i: (i, 0, 0)),
                  pl.BlockSpec((1, NVIS, 256), lambda i: (i, 0, 0)),
                  pl.BlockSpec((256, H), lambda i: (0, 0)),
                  pl.BlockSpec((1, H), lambda i: (0, 0))],
        out_specs=pl.BlockSpec((1, ROW_OUT, H), lambda i: (i, 0, 0)),
        out_shape=jax.ShapeDtypeStruct((N_PAGES, ROW_OUT, H), jnp.float32),
        input_output_aliases={0: 0},
    )(out_sc, patches, W, b.reshape(1, H))


def _sc_embed(shared, x_t, y_t, pe, ids, xi, yi, vxi, vyi, oidx, tidx,
              pe_i):
    mesh = plsc.VectorSubcoreMesh(core_axis_name="c", subcore_axis_name="s")

    @functools.partial(
        pl.kernel, mesh=mesh,
        out_type=jax.ShapeDtypeStruct((N_PAGES * ROW_OUT, H), jnp.float32),
        compiler_params=pltpu.CompilerParams(use_tc_tiling_on_sc=True),
        scratch_types=[
            pltpu.VMEM((256,), jnp.int32),
            pltpu.VMEM((768,), jnp.int32),
            pltpu.VMEM((768,), jnp.int32),
            pltpu.VMEM((512,), jnp.int32),
            pltpu.VMEM((8, 8), jnp.int32),
            pltpu.VMEM((8,), jnp.int32),
            pltpu.VMEM((CH, H), jnp.float32),
            pltpu.VMEM((CH, H), jnp.float32),
            pltpu.VMEM((2 * CH, H), jnp.float32),
            pltpu.VMEM((2 * CH, H), jnp.float32),
            pltpu.VMEM((2 * CH, H), jnp.float32),
            pltpu.VMEM((2 * CH, H), jnp.float32),
            pltpu.VMEM((2 * CH, H), jnp.float32),
            pltpu.VMEM((4, H), jnp.float32),
            pltpu.SemaphoreType.DMA,
            pltpu.SemaphoreType.DMA,
            pltpu.SemaphoreType.DMA,
            pltpu.SemaphoreType.DMA,
        ],
    )
    def k(shared_h, xt_h, yt_h, pe_h, ids_h, xi_h, yi_h, vxi_h, vyi_h,
          oidx_h, tidx_h, pei_h,
          out_h,
          ids_v, xi_v, yi_v, oidx_v, tidx_v, pe_idx_v,
          s0, s1, x0, x1, y0, y1, ob, pe_v,
          sem_g0, sem_g1, sem_s, sem_c):
        n = lax.axis_index("s")
        h = lax.axis_index("c")
        w = n * 2 + h
        tok0 = n * SEQ + h * 256
        p4 = n % 4

        S = (s0, s1)
        X = (x0, x1)
        Y = (y0, y1)
        SEM_G = (sem_g0, sem_g1)

        pltpu.sync_copy(ids_h.at[pl.ds(tok0, 256)], ids_v)
        pltpu.sync_copy(xi_h.at[pl.ds(2 * tok0, 512)], xi_v.at[pl.ds(0, 512)])
        pltpu.sync_copy(yi_h.at[pl.ds(2 * tok0, 512)], yi_v.at[pl.ds(0, 512)])
        pltpu.sync_copy(vxi_h.at[w], xi_v.at[pl.ds(512, 256)])
        pltpu.sync_copy(vyi_h.at[w], yi_v.at[pl.ds(512, 256)])
        pltpu.sync_copy(oidx_h.at[w], oidx_v)
        pltpu.sync_copy(tidx_h.at[w], tidx_v)
        pltpu.sync_copy(pei_h, pe_idx_v)

        vg = pltpu.async_copy(pe_h.at[pe_idx_v.at[pl.ds(0, 4)]], pe_v, sem_c)

        def fire(ci, p):
            off = ci * CH
            pltpu.async_copy(shared_h.at[ids_v.at[pl.ds(off, CH)]],
                             S[p], SEM_G[p])
            pltpu.async_copy(xt_h.at[xi_v.at[pl.ds(2 * off, 2 * CH)]],
                             X[p], SEM_G[p])
            pltpu.async_copy(yt_h.at[yi_v.at[pl.ds(2 * off, 2 * CH)]],
                             Y[p], SEM_G[p])

        def vis_fire(wi, p):
            pltpu.async_copy(xt_h.at[xi_v.at[pl.ds(512 + wi * 16, 16)]],
                             X[p], SEM_G[p])
            pltpu.async_copy(yt_h.at[yi_v.at[pl.ds(512 + wi * 16, 16)]],
                             Y[p], SEM_G[p])

        def wait_g(p):
            pltpu.make_async_copy(shared_h.at[pl.ds(0, CH)], S[p],
                                  SEM_G[p]).wait()
            pltpu.make_async_copy(xt_h.at[pl.ds(0, 2 * CH)], X[p],
                                  SEM_G[p]).wait()
            pltpu.make_async_copy(yt_h.at[pl.ds(0, 2 * CH)], Y[p],
                                  SEM_G[p]).wait()

        def wait_g_vis(p):
            pltpu.make_async_copy(xt_h.at[pl.ds(0, 2 * CH)], X[p],
                                  SEM_G[p]).wait()
            pltpu.make_async_copy(yt_h.at[pl.ds(0, 2 * CH)], Y[p],
                                  SEM_G[p]).wait()

        def wait_st():
            pltpu.make_async_copy(ob, out_h.at[pl.ds(0, 2 * CH)],
                                  sem_s).wait()

        def sum_rows(p):
            sp, xp, yp = S[p], X[p], Y[p]

            def rbody(r, carry):
                def load_pair(g):
                    sl0 = pl.ds(g * NLANE, NLANE)
                    sl1 = pl.ds((g + 1) * NLANE, NLANE)
                    return (sp[r, sl0], sp[r, sl1],
                            xp[2 * r, sl0], xp[2 * r, sl1],
                            xp[2 * r + 1, sl0], xp[2 * r + 1, sl1],
                            yp[2 * r, sl0], yp[2 * r, sl1],
                            yp[2 * r + 1, sl0], yp[2 * r + 1, sl1])

                def do_adds(g, t):
                    s_0, s_1, xa0, xa1, xb0, xb1, ya0, ya1, yb0, yb1 = t
                    sl0 = pl.ds(g * NLANE, NLANE)
                    sl1 = pl.ds((g + 1) * NLANE, NLANE)
                    a0 = s_0 + xa0
                    a1 = s_1 + xa1
                    b0_ = xb0 + ya0
                    b1_ = xb1 + ya1
                    ob[p * CH + r, sl0] = (a0 + yb0) + b0_
                    ob[p * CH + r, sl1] = (a1 + yb1) + b1_

                t = load_pair(0)
                for g in range(0, NCOL, 2):
                    nt = load_pair(g + 2) if g + 2 < NCOL else None
                    do_adds(g, t)
                    t = nt
                return carry
            lax.fori_loop(0, CH, rbody, 0)

        def sum_rows_vis(p):
            xp, yp = X[p], Y[p]

            def rbody(r, carry):
                def load_pair(g):
                    sl0 = pl.ds(g * NLANE, NLANE)
                    sl1 = pl.ds((g + 1) * NLANE, NLANE)
                    return (xp[2 * r, sl0], xp[2 * r, sl1],
                            xp[2 * r + 1, sl0], xp[2 * r + 1, sl1],
                            yp[2 * r, sl0], yp[2 * r, sl1],
                            yp[2 * r + 1, sl0], yp[2 * r + 1, sl1])

                def do_adds(g, t):
                    xa0, xa1, xb0, xb1, ya0, ya1, yb0, yb1 = t
                    sl0 = pl.ds(g * NLANE, NLANE)
                    sl1 = pl.ds((g + 1) * NLANE, NLANE)
                    a0 = xa0 + xb0
                    a1 = xa1 + xb1
                    b0_ = ya0 + yb0
                    b1_ = ya1 + yb1
                    ob[p * CH + r, sl0] = a0 + b0_
                    ob[p * CH + r, sl1] = a1 + b1_

                t = load_pair(0)
                for g in range(0, NCOL, 2):
                    nt = load_pair(g + 2) if g + 2 < NCOL else None
                    do_adds(g, t)
                    t = nt
                return carry
            lax.fori_loop(0, CH, rbody, 0)

        def pe_add(lo, hi):
            def rbody(r, carry):
                for g in range(NCOL):
                    sl = pl.ds(g * NLANE, NLANE)
                    ob[r, sl] = ob[r, sl] + pe_v[p4, sl]
                return carry
            lax.fori_loop(lo, hi, rbody, 0)

        def store_pair(idx_off):
            reg = oidx_v[pl.ds(idx_off, 16)]
            pltpu.async_copy(ob, out_h.at[reg], sem_s)

        fire(0, 0)
        fire(1, 1)

        def pair(j, carry):
            wait_g(0)
            pl.when(j >= 1)(lambda: wait_st())
            sum_rows(0)
            pl.when(j < 15)(lambda: fire(2 * j + 2, 0))
            wait_g(1)
            sum_rows(1)
            pl.when(j < 15)(lambda: fire(2 * j + 3, 1))
            pl.when(jnp.logical_and(j == 15, h == 1))(lambda: pe_add(6, 16))
            store_pair(j * 16)
            return carry
        lax.fori_loop(0, 16, pair, 0)

        vg.wait()
        wait_st()
        vis_fire(0, 0)
        vis_fire(1, 1)

        def vpair(j, carry):
            wait_g_vis(0)
            pl.when(j >= 1)(lambda: wait_st())
            sum_rows_vis(0)
            pl.when(j < 5)(lambda: vis_fire(2 * j + 2, 0))
            wait_g_vis(1)
            sum_rows_vis(1)
            pl.when(j < 5)(lambda: vis_fire(2 * j + 3, 1))
            store_pair(256 + j * 16)
            return carry
        lax.fori_loop(0, 6, vpair, 0)

        pltpu.async_copy(xt_h.at[xi_v.at[pl.ds(704, 16)]], X[0], SEM_G[0])
        pltpu.async_copy(yt_h.at[yi_v.at[pl.ds(704, 16)]], Y[0], SEM_G[0])
        pltpu.make_async_copy(xt_h.at[pl.ds(0, 2 * CH)], X[0], SEM_G[0]).wait()
        pltpu.make_async_copy(yt_h.at[pl.ds(0, 2 * CH)], Y[0], SEM_G[0]).wait()
        wait_st()

        for r in range(CH):
            for g in range(0, NCOL, 2):
                sl0 = pl.ds(g * NLANE, NLANE)
                sl1 = pl.ds((g + 1) * NLANE, NLANE)
                a0 = x0[2 * r, sl0] + x0[2 * r + 1, sl0]
                a1 = x0[2 * r, sl1] + x0[2 * r + 1, sl1]
                b0_ = y0[2 * r, sl0] + y0[2 * r + 1, sl0]
                b1_ = y0[2 * r, sl1] + y0[2 * r + 1, sl1]
                ob[r, sl0] = a0 + b0_
                ob[r, sl1] = a1 + b1_

        pltpu.async_copy(ob.at[pl.ds(0, CH)], out_h.at[tidx_v.at[0]], sem_s)
        pltpu.make_async_copy(ob.at[pl.ds(0, CH)], out_h.at[pl.ds(0, CH)],
                              sem_s).wait()

    return k(shared, x_t, y_t, pe, ids, xi, yi, vxi, vyi, oidx, tidx,
             pe_i)


def kernel(input_ids, boxes, images, shared, x_table, y_table, W_vis, b_vis):
    ids = input_ids.reshape(-1).astype(jnp.int32)
    bf = boxes.reshape(-1, 4).astype(jnp.int32)
    xi = bf[:, (0, 2)].reshape(-1)
    yi = bf[:, (1, 3)].reshape(-1)
    imgs = images.reshape(-1, IMG_SIDE, IMG_SIDE)
    patches = imgs.reshape(-1, GRID, 16, GRID, 16).transpose(0, 1, 3, 2, 4)
    patches = patches.reshape(N_PAGES, NVIS, 256)
    pe = jnp.asarray(_pe_table())
    vxi, vyi = _grid_vis_idx()
    oidx, tidx = _out_row_idx()
    pe_i = jnp.asarray(np.arange(8, dtype=np.int32))
    out_sc = _sc_embed(shared, x_table, y_table, pe, ids, xi, yi,
                       jnp.asarray(vxi), jnp.asarray(vyi), jnp.asarray(oidx),
                       jnp.asarray(tidx), pe_i)
    return _visual_add(out_sc.reshape(N_PAGES, ROW_OUT, H), patches,
                       W_vis, b_vis)
